# Initial kernel scaffold; baseline (speedup 1.0000x reference)
import jax
import jax.numpy as jnp
from jax.experimental import pallas as pl


def kernel(x, edge_index, edge_weight, W, b):
    raise NotImplementedError("write your pallas kernel here")


# speedup vs baseline: 21.8072x; 21.8072x over previous
import functools

import jax
import jax.numpy as jnp
from jax import lax
from jax.experimental import pallas as pl
from jax.experimental.pallas import tpu as pltpu, tpu_sc as plsc

N = 10000
E = 320000
D = 128
N_PAD = 10240
NC, NS = 2, 16
NW = NC * NS
EPT = E // NW
K = 80
CH = EPT // K
RPT = N_PAD // NS
L = 16


def _sc_mesh():
    return plsc.VectorSubcoreMesh(core_axis_name="c", subcore_axis_name="s")


def _splat(vec, lane):
    idx = jnp.full((L, 1), lane, jnp.int32)
    dnums = lax.GatherDimensionNumbers(
        offset_dims=(), collapsed_slice_dims=(0,), start_index_map=(0,))
    return lax.gather(vec, idx, dnums, (1,),
                      mode=lax.GatherScatterMode.PROMISE_IN_BOUNDS)


def _deg_body(ni2_hbm, no2_hbm, ew_hbm, din_out, dout_out,
              ni_v, no_v, ew_v, z_v, din_sh, dout_sh):
    cid = lax.axis_index("c")
    sid = lax.axis_index("s")
    wid = cid * NS + sid

    pltpu.sync_copy(ni2_hbm.at[wid], ni_v)
    pltpu.sync_copy(no2_hbm.at[wid], no_v)
    pltpu.sync_copy(ew_hbm.at[pl.ds(wid * EPT, EPT)], ew_v)

    def _z(i, carry):
        z_v[pl.ds(i * L, L)] = jnp.zeros((L,), jnp.float32)
        return carry
    lax.fori_loop(0, RPT // L, _z, 0)
    pltpu.sync_copy(z_v, din_sh.at[pl.ds(sid * RPT, RPT)])
    pltpu.sync_copy(z_v, dout_sh.at[pl.ds(sid * RPT, RPT)])
    plsc.subcore_barrier()

    def _chunk(c, carry):
        src = ew_v.at[pl.ds(c * K, K)]
        pltpu.sync_copy(src, din_sh.at[ni_v.at[c]], add=True)
        pltpu.sync_copy(src, dout_sh.at[no_v.at[c]], add=True)
        return carry
    lax.fori_loop(0, CH, _chunk, 0)
    plsc.subcore_barrier()

    pltpu.sync_copy(din_sh.at[pl.ds(sid * RPT, RPT)], z_v)
    pltpu.sync_copy(z_v, din_out.at[pl.ds(cid * N_PAD + sid * RPT, RPT)])
    pltpu.sync_copy(dout_sh.at[pl.ds(sid * RPT, RPT)], z_v)
    pltpu.sync_copy(z_v, dout_out.at[pl.ds(cid * N_PAD + sid * RPT, RPT)])


@jax.jit
def _sc_degrees(ni2, no2, ew):
    return pl.kernel(
        _deg_body,
        out_type=[jax.ShapeDtypeStruct((NC * N_PAD,), jnp.float32),
                  jax.ShapeDtypeStruct((NC * N_PAD,), jnp.float32)],
        mesh=_sc_mesh(),
        scratch_types=[
            pltpu.VMEM((CH, K), jnp.int32),
            pltpu.VMEM((CH, K), jnp.int32),
            pltpu.VMEM((EPT,), jnp.float32),
            pltpu.VMEM((RPT,), jnp.float32),
            pltpu.VMEM_SHARED((N_PAD,), jnp.float32),
            pltpu.VMEM_SHARED((N_PAD,), jnp.float32),
        ],
    )(ni2, no2, ew)


def _y_body(dinp_ref, x_ref, y_ref):
    deg = dinp_ref[0, :] + dinp_ref[1, :] + 1.0
    rin = lax.rsqrt(deg)
    y_ref[...] = rin[:, None] * x_ref[...]


@jax.jit
def _tc_scale_x(din_p, x_pad):
    blk = 1024
    grid = N_PAD // blk
    return pl.pallas_call(
        _y_body,
        grid=(grid,),
        in_specs=[
            pl.BlockSpec((NC, blk), lambda i: (0, i)),
            pl.BlockSpec((blk, D), lambda i: (i, 0)),
        ],
        out_specs=pl.BlockSpec((blk, D), lambda i: (i, 0)),
        out_shape=jax.ShapeDtypeStruct((N_PAD, D), jnp.float32),
    )(din_p, x_pad)


def _edge_body(ni_hbm, no2_hbm, ew_hbm, y_hbm, acc_out,
               ni_v, no_v, ew_v, rows_v, gsem, acc_sh):
    cid = lax.axis_index("c")
    sid = lax.axis_index("s")
    wid = cid * NS + sid

    pltpu.sync_copy(ni_hbm.at[pl.ds(wid * EPT, EPT)], ni_v)
    pltpu.sync_copy(no2_hbm.at[wid], no_v)
    pltpu.sync_copy(ew_hbm.at[pl.ds(wid * EPT, EPT)], ew_v)

    def _zr(r, carry):
        for q in range(D // L):
            rows_v[r, pl.ds(q * L, L)] = jnp.zeros((L,), jnp.float32)
        return carry
    lax.fori_loop(0, K, _zr, 0)
    for kk in range(RPT // K):
        pltpu.sync_copy(rows_v, acc_sh.at[pl.ds(sid * RPT + kk * K, K)])
    plsc.subcore_barrier()

    def _chunk(c, carry):
        pltpu.async_copy(y_hbm.at[ni_v.at[pl.ds(c * K, K)]], rows_v,
                         gsem).wait()

        def _grp(g, gcarry):
            ewg = ew_v[pl.ds(c * K + g * L, L)]
            for lane in range(L):
                w = _splat(ewg, lane)
                r = g * L + lane
                for q in range(D // L):
                    sl = pl.ds(q * L, L)
                    rows_v[r, sl] = rows_v[r, sl] * w
            return gcarry
        lax.fori_loop(0, K // L, _grp, 0)

        pltpu.sync_copy(rows_v, acc_sh.at[no_v.at[c]], add=True)
        return carry
    lax.fori_loop(0, CH, _chunk, 0)
    plsc.subcore_barrier()

    for kk in range(RPT // K):
        sl = pl.ds(sid * RPT + kk * K, K)
        pltpu.sync_copy(acc_sh.at[sl], rows_v)
        pltpu.sync_copy(rows_v, acc_out.at[cid, sl])


@jax.jit
def _sc_edge_pass(ni, no2, ew, y):
    return pl.kernel(
        _edge_body,
        out_type=jax.ShapeDtypeStruct((NC, N_PAD, D), jnp.float32),
        mesh=_sc_mesh(),
        scratch_types=[
            pltpu.VMEM((EPT,), jnp.int32),
            pltpu.VMEM((CH, K), jnp.int32),
            pltpu.VMEM((EPT,), jnp.float32),
            pltpu.VMEM((K, D), jnp.float32),
            pltpu.SemaphoreType.DMA,
            pltpu.VMEM_SHARED((N_PAD, D), jnp.float32),
        ],
    )(ni, no2, ew, y)


def _out_body(accp_ref, y_ref, doutp_ref, w_ref, b_ref, o_ref):
    deg = doutp_ref[0, :] + doutp_ref[1, :] + 1.0
    rout = lax.rsqrt(deg)
    u = (accp_ref[0] + accp_ref[1] + y_ref[...]) * rout[:, None]
    acc = lax.dot_general(u, w_ref[...], (((1,), (1,)), ((), ())),
                          preferred_element_type=jnp.float32)
    o_ref[...] = jnp.maximum(acc + b_ref[...], 0.0)


@jax.jit
def _tc_combine(acc_p, y, dout_p, W, b2):
    blk = 1024
    grid = N_PAD // blk
    return pl.pallas_call(
        _out_body,
        grid=(grid,),
        in_specs=[
            pl.BlockSpec((NC, blk, D), lambda i: (0, i, 0)),
            pl.BlockSpec((blk, D), lambda i: (i, 0)),
            pl.BlockSpec((NC, blk), lambda i: (0, i)),
            pl.BlockSpec((D, D), lambda i: (0, 0)),
            pl.BlockSpec((1, D), lambda i: (0, 0)),
        ],
        out_specs=pl.BlockSpec((blk, D), lambda i: (i, 0)),
        out_shape=jax.ShapeDtypeStruct((N_PAD, D), jnp.float32),
    )(acc_p, y, dout_p, W, b2)


def kernel(x, edge_index, edge_weight, W, b):
    ni = edge_index[:, 0]
    no = edge_index[:, 1]
    ni2 = ni.reshape(NW, CH, K)
    no2 = no.reshape(NW, CH, K)
    x_pad = jnp.pad(x, ((0, N_PAD - N), (0, 0)))
    b2 = b.reshape(1, D)

    din_f, dout_f = _sc_degrees(ni2, no2, edge_weight)
    din_p = din_f.reshape(NC, N_PAD)
    dout_p = dout_f.reshape(NC, N_PAD)
    y = _tc_scale_x(din_p, x_pad)
    acc_p = _sc_edge_pass(ni, no2, edge_weight, y)
    out = _tc_combine(acc_p, y, dout_p, W, b2)
    return out[:N]


# speedup vs baseline: 34.3908x; 1.5770x over previous
import jax
import jax.numpy as jnp
from jax import lax
from jax.experimental import pallas as pl
from jax.experimental.pallas import tpu as pltpu, tpu_sc as plsc

N = 10000
E = 320000
D = 128
N_PAD = 10240
NC, NS = 2, 16
NW = NC * NS
RPT = N_PAD // NS
L = 16

KD = 80
CHD = 125

KE = 64
CHE = 160
EPTE = KE * CHE
PADE = EPTE - E // NW
NBUF = 5


def _sc_mesh():
    return plsc.VectorSubcoreMesh(core_axis_name="c", subcore_axis_name="s")


def _splat(vec, lane):
    idx = jnp.full((L, 1), lane, jnp.int32)
    dnums = lax.GatherDimensionNumbers(
        offset_dims=(), collapsed_slice_dims=(0,), start_index_map=(0,))
    return lax.gather(vec, idx, dnums, (1,),
                      mode=lax.GatherScatterMode.PROMISE_IN_BOUNDS)


def _deg_body(ni2_hbm, no2_hbm, ew_hbm, din_out, dout_out,
              ni_v, no_v, ew_v, z_v, din_sh, dout_sh):
    cid = lax.axis_index("c")
    sid = lax.axis_index("s")
    wid = cid * NS + sid
    ept = E // NW

    pltpu.sync_copy(ni2_hbm.at[wid], ni_v)
    pltpu.sync_copy(no2_hbm.at[wid], no_v)
    pltpu.sync_copy(ew_hbm.at[pl.ds(wid * ept, ept)], ew_v)

    def _z(i, carry):
        z_v[pl.ds(i * L, L)] = jnp.zeros((L,), jnp.float32)
        return carry
    lax.fori_loop(0, RPT // L, _z, 0)
    pltpu.sync_copy(z_v, din_sh.at[pl.ds(sid * RPT, RPT)])
    pltpu.sync_copy(z_v, dout_sh.at[pl.ds(sid * RPT, RPT)])
    plsc.subcore_barrier()

    def _chunk(c, carry):
        src = ew_v.at[pl.ds(c * KD, KD)]
        pltpu.sync_copy(src, din_sh.at[ni_v.at[c]], add=True)
        pltpu.sync_copy(src, dout_sh.at[no_v.at[c]], add=True)
        return carry
    lax.fori_loop(0, CHD, _chunk, 0)
    plsc.subcore_barrier()

    pltpu.sync_copy(din_sh.at[pl.ds(sid * RPT, RPT)], z_v)
    pltpu.sync_copy(z_v, din_out.at[pl.ds(cid * N_PAD + sid * RPT, RPT)])
    pltpu.sync_copy(dout_sh.at[pl.ds(sid * RPT, RPT)], z_v)
    pltpu.sync_copy(z_v, dout_out.at[pl.ds(cid * N_PAD + sid * RPT, RPT)])


@jax.jit
def _sc_degrees(ni2, no2, ew):
    return pl.kernel(
        _deg_body,
        out_type=[jax.ShapeDtypeStruct((NC * N_PAD,), jnp.float32),
                  jax.ShapeDtypeStruct((NC * N_PAD,), jnp.float32)],
        mesh=_sc_mesh(),
        scratch_types=[
            pltpu.VMEM((CHD, KD), jnp.int32),
            pltpu.VMEM((CHD, KD), jnp.int32),
            pltpu.VMEM((E // NW,), jnp.float32),
            pltpu.VMEM((RPT,), jnp.float32),
            pltpu.VMEM_SHARED((N_PAD,), jnp.float32),
            pltpu.VMEM_SHARED((N_PAD,), jnp.float32),
        ],
    )(ni2, no2, ew)


def _y_body(dinp_ref, x_ref, y_ref):
    deg = dinp_ref[0, :] + dinp_ref[1, :] + 1.0
    rin = lax.rsqrt(deg)
    y_ref[...] = rin[:, None] * x_ref[...]


@jax.jit
def _tc_scale_x(din_p, x_pad):
    blk = 1024
    return pl.pallas_call(
        _y_body,
        grid=(N_PAD // blk,),
        in_specs=[
            pl.BlockSpec((NC, blk), lambda i: (0, i)),
            pl.BlockSpec((blk, D), lambda i: (i, 0)),
        ],
        out_specs=pl.BlockSpec((blk, D), lambda i: (i, 0)),
        out_shape=jax.ShapeDtypeStruct((N_PAD, D), jnp.float32),
    )(din_p, x_pad)


def _edge_body(ni_hbm, no_hbm, ew_hbm, y_hbm, acc_out,
               nir, nor, ewr, bufs, isems, gsems, ssems, acc_sh):
    cid = lax.axis_index("c")
    sid = lax.axis_index("s")
    wid = cid * NS + sid
    base = wid * EPTE

    def _zr(r, carry):
        for q in range(D // L):
            bufs[0, r, pl.ds(q * L, L)] = jnp.zeros((L,), jnp.float32)
        return carry
    lax.fori_loop(0, KE, _zr, 0)
    for kk in range(RPT // KE):
        pltpu.sync_copy(bufs.at[0], acc_sh.at[pl.ds(sid * RPT + kk * KE, KE)])
    plsc.subcore_barrier()

    def _idx_start(t, b):
        sl = pl.ds(base + t * KE, KE)
        pltpu.async_copy(ni_hbm.at[sl], nir.at[b], isems.at[b])
        pltpu.async_copy(no_hbm.at[sl], nor.at[b], isems.at[b])
        pltpu.async_copy(ew_hbm.at[sl], ewr.at[b], isems.at[b])

    def _idx_wait(t, b):
        sl = pl.ds(base + t * KE, KE)
        pltpu.make_async_copy(ni_hbm.at[sl], nir.at[b], isems.at[b]).wait()
        pltpu.make_async_copy(no_hbm.at[sl], nor.at[b], isems.at[b]).wait()
        pltpu.make_async_copy(ew_hbm.at[sl], ewr.at[b], isems.at[b]).wait()

    def _gather_start(b):
        pltpu.async_copy(y_hbm.at[nir.at[b]], bufs.at[b], gsems.at[b])

    def _gather_wait(b):
        pltpu.make_async_copy(y_hbm.at[nir.at[b]], bufs.at[b],
                              gsems.at[b]).wait()

    def _scatter_start(b):
        pltpu.async_copy(bufs.at[b], acc_sh.at[nor.at[b]], ssems.at[b],
                         add=True)

    def _scatter_wait(b):
        pltpu.make_async_copy(bufs.at[b], acc_sh.at[nor.at[b]],
                              ssems.at[b]).wait()

    def _scale(t, b):
        def _grp(g, gcarry):
            ewg = ewr[b, pl.ds(g * L, L)]
            for lane in range(L):
                w = _splat(ewg, lane)
                r = g * L + lane
                for q in range(D // L):
                    sl = pl.ds(q * L, L)
                    bufs[b, r, sl] = bufs[b, r, sl] * w
            return gcarry
        lax.fori_loop(0, KE // L, _grp, 0)

    for b in range(NBUF):
        _idx_start(b, b)
    for b in range(NBUF - 2):
        _idx_wait(b, b)
        _gather_start(b)

    def _slot(g, carry):
        for b in range(NBUF):
            t = g * NBUF + b
            pb = (b + NBUF - 1) % NBUF
            b3 = (b + NBUF - 2) % NBUF

            @pl.when(t > 0)
            def _():
                _scatter_wait(pb)

            @pl.when(jnp.logical_and(t > 0, t + NBUF - 1 < CHE))
            def _():
                _idx_start(t + NBUF - 1, pb)

            @pl.when(t + NBUF - 2 < CHE)
            def _():
                _idx_wait(t + NBUF - 2, b3)
                _gather_start(b3)

            _gather_wait(b)
            _scale(t, b)
            _scatter_start(b)
        return carry
    lax.fori_loop(0, CHE // NBUF, _slot, 0)
    _scatter_wait((CHE - 1) % NBUF)
    plsc.subcore_barrier()

    for kk in range(RPT // KE):
        sl = pl.ds(sid * RPT + kk * KE, KE)
        pltpu.sync_copy(acc_sh.at[sl], bufs.at[0])
        pltpu.sync_copy(bufs.at[0], acc_out.at[cid, sl])


@jax.jit
def _sc_edge_pass(ni, no, ew, y):
    return pl.kernel(
        _edge_body,
        out_type=jax.ShapeDtypeStruct((NC, N_PAD, D), jnp.float32),
        mesh=_sc_mesh(),
        scratch_types=[
            pltpu.VMEM((NBUF, KE), jnp.int32),
            pltpu.VMEM((NBUF, KE), jnp.int32),
            pltpu.VMEM((NBUF, KE), jnp.float32),
            pltpu.VMEM((NBUF, KE, D), jnp.float32),
            pltpu.SemaphoreType.DMA((NBUF,)),
            pltpu.SemaphoreType.DMA((NBUF,)),
            pltpu.SemaphoreType.DMA((NBUF,)),
            pltpu.VMEM_SHARED((N_PAD, D), jnp.float32),
        ],
    )(ni, no, ew, y)


def _out_body(accp_ref, y_ref, doutp_ref, w_ref, b_ref, o_ref):
    deg = doutp_ref[0, :] + doutp_ref[1, :] + 1.0
    rout = lax.rsqrt(deg)
    u = (accp_ref[0] + accp_ref[1] + y_ref[...]) * rout[:, None]
    acc = lax.dot_general(u, w_ref[...], (((1,), (1,)), ((), ())),
                          preferred_element_type=jnp.float32)
    o_ref[...] = jnp.maximum(acc + b_ref[...], 0.0)


@jax.jit
def _tc_combine(acc_p, y, dout_p, W, b2):
    blk = 1024
    return pl.pallas_call(
        _out_body,
        grid=(N_PAD // blk,),
        in_specs=[
            pl.BlockSpec((NC, blk, D), lambda i: (0, i, 0)),
            pl.BlockSpec((blk, D), lambda i: (i, 0)),
            pl.BlockSpec((NC, blk), lambda i: (0, i)),
            pl.BlockSpec((D, D), lambda i: (0, 0)),
            pl.BlockSpec((1, D), lambda i: (0, 0)),
        ],
        out_specs=pl.BlockSpec((blk, D), lambda i: (i, 0)),
        out_shape=jax.ShapeDtypeStruct((N_PAD, D), jnp.float32),
    )(acc_p, y, dout_p, W, b2)


def kernel(x, edge_index, edge_weight, W, b):
    ni = edge_index[:, 0]
    no = edge_index[:, 1]
    ept = E // NW

    ni2 = ni.reshape(NW, CHD, KD)
    no2 = no.reshape(NW, CHD, KD)

    pad_src = (jnp.arange(PADE, dtype=jnp.int32) * 41) % N
    pad_dst = (jnp.arange(PADE, dtype=jnp.int32) * 41) % N_PAD
    ni_p = jnp.concatenate(
        [ni.reshape(NW, ept),
         jnp.broadcast_to(pad_src, (NW, PADE))], axis=1).reshape(-1)
    no_p = jnp.concatenate(
        [no.reshape(NW, ept),
         jnp.broadcast_to(pad_dst, (NW, PADE))], axis=1).reshape(-1)
    ew_p = jnp.concatenate(
        [edge_weight.reshape(NW, ept),
         jnp.zeros((NW, PADE), jnp.float32)], axis=1).reshape(-1)

    x_pad = jnp.pad(x, ((0, N_PAD - N), (0, 0)))
    b2 = b.reshape(1, D)

    din_f, dout_f = _sc_degrees(ni2, no2, edge_weight)
    din_p = din_f.reshape(NC, N_PAD)
    dout_p = dout_f.reshape(NC, N_PAD)
    y = _tc_scale_x(din_p, x_pad)
    acc_p = _sc_edge_pass(ni_p, no_p, ew_p, y)
    out = _tc_combine(acc_p, y, dout_p, W, b2)
    return out[:N]


# speedup vs baseline: 37.4285x; 1.0883x over previous
import jax
import jax.numpy as jnp
from jax import lax
from jax.experimental import pallas as pl
from jax.experimental.pallas import tpu as pltpu, tpu_sc as plsc

N = 10000
E = 320000
D = 128
N_PAD = 10240
NC, NS = 2, 16
NW = NC * NS
RPT = N_PAD // NS
L = 16

KD = 80
CHD = 125

KE = 80
CHE = 128
EPTE = KE * CHE
PADE = EPTE - E // NW
NBUF = 4


def _sc_mesh():
    return plsc.VectorSubcoreMesh(core_axis_name="c", subcore_axis_name="s")


def _splat(vec, lane):
    idx = jnp.full((L, 1), lane, jnp.int32)
    dnums = lax.GatherDimensionNumbers(
        offset_dims=(), collapsed_slice_dims=(0,), start_index_map=(0,))
    return lax.gather(vec, idx, dnums, (1,),
                      mode=lax.GatherScatterMode.PROMISE_IN_BOUNDS)


def _deg_body(ni2_hbm, no2_hbm, ew_hbm, din_out, dout_out,
              ni_v, no_v, ew_v, z_v, dsem, osem, din_sh, dout_sh):
    cid = lax.axis_index("c")
    sid = lax.axis_index("s")
    wid = cid * NS + sid
    ept = E // NW

    pltpu.sync_copy(ni2_hbm.at[wid], ni_v)
    pltpu.sync_copy(no2_hbm.at[wid], no_v)
    pltpu.sync_copy(ew_hbm.at[pl.ds(wid * ept, ept)], ew_v)

    def _z(i, carry):
        z_v[pl.ds(i * L, L)] = jnp.zeros((L,), jnp.float32)
        return carry
    lax.fori_loop(0, RPT // L, _z, 0)
    pltpu.sync_copy(z_v, din_sh.at[pl.ds(sid * RPT, RPT)])
    pltpu.sync_copy(z_v, dout_sh.at[pl.ds(sid * RPT, RPT)])
    plsc.subcore_barrier()

    GRP = 25
    def _grp_sc(g, carry):
        for j in range(GRP):
            c = g * GRP + j
            src = ew_v.at[pl.ds(c * KD, KD)]
            pltpu.async_copy(src, din_sh.at[ni_v.at[c]], dsem, add=True)
            pltpu.async_copy(src, dout_sh.at[no_v.at[c]], osem, add=True)
        for j in range(GRP):
            c = g * GRP + j
            src = ew_v.at[pl.ds(c * KD, KD)]
            pltpu.make_async_copy(src, din_sh.at[ni_v.at[c]], dsem).wait()
            pltpu.make_async_copy(src, dout_sh.at[no_v.at[c]], osem).wait()
        return carry
    lax.fori_loop(0, CHD // GRP, _grp_sc, 0)
    plsc.subcore_barrier()

    pltpu.sync_copy(din_sh.at[pl.ds(sid * RPT, RPT)], z_v)
    pltpu.sync_copy(z_v, din_out.at[pl.ds(cid * N_PAD + sid * RPT, RPT)])
    pltpu.sync_copy(dout_sh.at[pl.ds(sid * RPT, RPT)], z_v)
    pltpu.sync_copy(z_v, dout_out.at[pl.ds(cid * N_PAD + sid * RPT, RPT)])


@jax.jit
def _sc_degrees(ni2, no2, ew):
    return pl.kernel(
        _deg_body,
        out_type=[jax.ShapeDtypeStruct((NC * N_PAD,), jnp.float32),
                  jax.ShapeDtypeStruct((NC * N_PAD,), jnp.float32)],
        mesh=_sc_mesh(),
        scratch_types=[
            pltpu.VMEM((CHD, KD), jnp.int32),
            pltpu.VMEM((CHD, KD), jnp.int32),
            pltpu.VMEM((E // NW,), jnp.float32),
            pltpu.VMEM((RPT,), jnp.float32),
            pltpu.SemaphoreType.DMA,
            pltpu.SemaphoreType.DMA,
            pltpu.VMEM_SHARED((N_PAD,), jnp.float32),
            pltpu.VMEM_SHARED((N_PAD,), jnp.float32),
        ],
    )(ni2, no2, ew)


def _y_body(dinp_ref, x_ref, y_ref):
    deg = dinp_ref[0, :] + dinp_ref[1, :] + 1.0
    rin = lax.rsqrt(deg)
    y_ref[...] = rin[:, None] * x_ref[...]


@jax.jit
def _tc_scale_x(din_p, x_pad):
    blk = 1024
    return pl.pallas_call(
        _y_body,
        grid=(N_PAD // blk,),
        in_specs=[
            pl.BlockSpec((NC, blk), lambda i: (0, i)),
            pl.BlockSpec((blk, D), lambda i: (i, 0)),
        ],
        out_specs=pl.BlockSpec((blk, D), lambda i: (i, 0)),
        out_shape=jax.ShapeDtypeStruct((N_PAD, D), jnp.float32),
    )(din_p, x_pad)


def _edge_body(ni_hbm, no_hbm, ew_hbm, y_hbm, acc_out,
               nir, nor, ewr, bufs, isems, gsems, ssems, acc_sh):
    cid = lax.axis_index("c")
    sid = lax.axis_index("s")
    wid = cid * NS + sid
    base = wid * EPTE

    def _zr(r, carry):
        for q in range(D // L):
            bufs[0, r, pl.ds(q * L, L)] = jnp.zeros((L,), jnp.float32)
        return carry
    lax.fori_loop(0, KE, _zr, 0)
    for kk in range(RPT // KE):
        pltpu.sync_copy(bufs.at[0], acc_sh.at[pl.ds(sid * RPT + kk * KE, KE)])
    plsc.subcore_barrier()

    def _idx_start(t, b):
        sl = pl.ds(base + t * KE, KE)
        pltpu.async_copy(ni_hbm.at[sl], nir.at[b], isems.at[b])
        pltpu.async_copy(no_hbm.at[sl], nor.at[b], isems.at[b])
        pltpu.async_copy(ew_hbm.at[sl], ewr.at[b], isems.at[b])

    def _idx_wait(t, b):
        sl = pl.ds(base + t * KE, KE)
        pltpu.make_async_copy(ni_hbm.at[sl], nir.at[b], isems.at[b]).wait()
        pltpu.make_async_copy(no_hbm.at[sl], nor.at[b], isems.at[b]).wait()
        pltpu.make_async_copy(ew_hbm.at[sl], ewr.at[b], isems.at[b]).wait()

    def _gather_start(b):
        pltpu.async_copy(y_hbm.at[nir.at[b]], bufs.at[b], gsems.at[b])

    def _gather_wait(b):
        pltpu.make_async_copy(y_hbm.at[nir.at[b]], bufs.at[b],
                              gsems.at[b]).wait()

    def _scatter_start(b):
        pltpu.async_copy(bufs.at[b], acc_sh.at[nor.at[b]], ssems.at[b],
                         add=True)

    def _scatter_wait(b):
        pltpu.make_async_copy(bufs.at[b], acc_sh.at[nor.at[b]],
                              ssems.at[b]).wait()

    def _scale(t, b):
        def _grp(g, gcarry):
            ewg = ewr[b, pl.ds(g * L, L)]
            for lane in range(L):
                w = _splat(ewg, lane)
                r = g * L + lane
                for q in range(D // L):
                    sl = pl.ds(q * L, L)
                    bufs[b, r, sl] = bufs[b, r, sl] * w
            return gcarry
        lax.fori_loop(0, KE // L, _grp, 0)

    for b in range(NBUF):
        _idx_start(b, b)
    for b in range(NBUF - 2):
        _idx_wait(b, b)
        _gather_start(b)

    def _slot(g, carry):
        for b in range(NBUF):
            t = g * NBUF + b
            pb = (b + NBUF - 1) % NBUF
            b3 = (b + NBUF - 2) % NBUF

            @pl.when(t > 0)
            def _():
                _scatter_wait(pb)

            @pl.when(jnp.logical_and(t > 0, t + NBUF - 1 < CHE))
            def _():
                _idx_start(t + NBUF - 1, pb)

            @pl.when(t + NBUF - 2 < CHE)
            def _():
                _idx_wait(t + NBUF - 2, b3)
                _gather_start(b3)

            _gather_wait(b)
            _scale(t, b)
            _scatter_start(b)
        return carry
    lax.fori_loop(0, CHE // NBUF, _slot, 0)
    _scatter_wait((CHE - 1) % NBUF)
    plsc.subcore_barrier()

    for kk in range(RPT // KE):
        sl = pl.ds(sid * RPT + kk * KE, KE)
        pltpu.sync_copy(acc_sh.at[sl], bufs.at[0])
        pltpu.sync_copy(bufs.at[0], acc_out.at[cid, sl])


@jax.jit
def _sc_edge_pass(ni, no, ew, y):
    return pl.kernel(
        _edge_body,
        out_type=jax.ShapeDtypeStruct((NC, N_PAD, D), jnp.float32),
        mesh=_sc_mesh(),
        scratch_types=[
            pltpu.VMEM((NBUF, KE), jnp.int32),
            pltpu.VMEM((NBUF, KE), jnp.int32),
            pltpu.VMEM((NBUF, KE), jnp.float32),
            pltpu.VMEM((NBUF, KE, D), jnp.float32),
            pltpu.SemaphoreType.DMA((NBUF,)),
            pltpu.SemaphoreType.DMA((NBUF,)),
            pltpu.SemaphoreType.DMA((NBUF,)),
            pltpu.VMEM_SHARED((N_PAD, D), jnp.float32),
        ],
    )(ni, no, ew, y)


def _out_body(accp_ref, y_ref, doutp_ref, w_ref, b_ref, o_ref):
    deg = doutp_ref[0, :] + doutp_ref[1, :] + 1.0
    rout = lax.rsqrt(deg)
    u = (accp_ref[0] + accp_ref[1] + y_ref[...]) * rout[:, None]
    acc = lax.dot_general(u, w_ref[...], (((1,), (1,)), ((), ())),
                          preferred_element_type=jnp.float32)
    o_ref[...] = jnp.maximum(acc + b_ref[...], 0.0)


@jax.jit
def _tc_combine(acc_p, y, dout_p, W, b2):
    blk = 1024
    return pl.pallas_call(
        _out_body,
        grid=(N_PAD // blk,),
        in_specs=[
            pl.BlockSpec((NC, blk, D), lambda i: (0, i, 0)),
            pl.BlockSpec((blk, D), lambda i: (i, 0)),
            pl.BlockSpec((NC, blk), lambda i: (0, i)),
            pl.BlockSpec((D, D), lambda i: (0, 0)),
            pl.BlockSpec((1, D), lambda i: (0, 0)),
        ],
        out_specs=pl.BlockSpec((blk, D), lambda i: (i, 0)),
        out_shape=jax.ShapeDtypeStruct((N_PAD, D), jnp.float32),
    )(acc_p, y, dout_p, W, b2)


def kernel(x, edge_index, edge_weight, W, b):
    ni = edge_index[:, 0]
    no = edge_index[:, 1]
    ept = E // NW

    ni2 = ni.reshape(NW, CHD, KD)
    no2 = no.reshape(NW, CHD, KD)

    pad_src = (jnp.arange(PADE, dtype=jnp.int32) * 41) % N
    pad_dst = (jnp.arange(PADE, dtype=jnp.int32) * 41) % N_PAD
    ni_p = jnp.concatenate(
        [ni.reshape(NW, ept),
         jnp.broadcast_to(pad_src, (NW, PADE))], axis=1).reshape(-1)
    no_p = jnp.concatenate(
        [no.reshape(NW, ept),
         jnp.broadcast_to(pad_dst, (NW, PADE))], axis=1).reshape(-1)
    ew_p = jnp.concatenate(
        [edge_weight.reshape(NW, ept),
         jnp.zeros((NW, PADE), jnp.float32)], axis=1).reshape(-1)

    x_pad = jnp.pad(x, ((0, N_PAD - N), (0, 0)))
    b2 = b.reshape(1, D)

    din_f, dout_f = _sc_degrees(ni2, no2, edge_weight)
    din_p = din_f.reshape(NC, N_PAD)
    dout_p = dout_f.reshape(NC, N_PAD)
    y = _tc_scale_x(din_p, x_pad)
    acc_p = _sc_edge_pass(ni_p, no_p, ew_p, y)
    out = _tc_combine(acc_p, y, dout_p, W, b2)
    return out[:N]


# speedup vs baseline: 37.4750x; 1.0012x over previous
import jax
import jax.numpy as jnp
from jax import lax
from jax.experimental import pallas as pl
from jax.experimental.pallas import tpu as pltpu, tpu_sc as plsc

N = 10000
E = 320000
D = 128
N_PAD = 10240
NC, NS = 2, 16
NW = NC * NS
RPT = N_PAD // NS
L = 16

KD = 80
CHD = 125

KE = 80
CHE = 128
EPTE = KE * CHE
PADE = EPTE - E // NW
NBUF = 4


def _sc_mesh():
    return plsc.VectorSubcoreMesh(core_axis_name="c", subcore_axis_name="s")


def _splat(vec, lane):
    idx = jnp.full((L, 1), lane, jnp.int32)
    dnums = lax.GatherDimensionNumbers(
        offset_dims=(), collapsed_slice_dims=(0,), start_index_map=(0,))
    return lax.gather(vec, idx, dnums, (1,),
                      mode=lax.GatherScatterMode.PROMISE_IN_BOUNDS)


def _rsqrt_nr(d):
    i = lax.bitcast_convert_type(d, jnp.int32)
    i = jnp.int32(0x5F3759DF) - lax.shift_right_logical(i, 1)
    r = lax.bitcast_convert_type(i, jnp.float32)
    for _ in range(3):
        r = r * (1.5 - 0.5 * d * r * r)
    return r


def _deg_body(ni2_hbm, no2_hbm, ew_hbm, din_out, dout_out,
              ni_v, no_v, ew_v, z_v, dsem, osem, din_sh, dout_sh):
    cid = lax.axis_index("c")
    sid = lax.axis_index("s")
    wid = cid * NS + sid
    ept = E // NW

    pltpu.sync_copy(ni2_hbm.at[wid], ni_v)
    pltpu.sync_copy(no2_hbm.at[wid], no_v)
    pltpu.sync_copy(ew_hbm.at[pl.ds(wid * ept, ept)], ew_v)

    def _z(i, carry):
        z_v[pl.ds(i * L, L)] = jnp.zeros((L,), jnp.float32)
        return carry
    lax.fori_loop(0, RPT // L, _z, 0)
    pltpu.sync_copy(z_v, din_sh.at[pl.ds(sid * RPT, RPT)])
    pltpu.sync_copy(z_v, dout_sh.at[pl.ds(sid * RPT, RPT)])
    plsc.subcore_barrier()

    GRP = 25
    def _grp_sc(g, carry):
        for j in range(GRP):
            c = g * GRP + j
            src = ew_v.at[pl.ds(c * KD, KD)]
            pltpu.async_copy(src, din_sh.at[ni_v.at[c]], dsem, add=True)
            pltpu.async_copy(src, dout_sh.at[no_v.at[c]], osem, add=True)
        for j in range(GRP):
            c = g * GRP + j
            src = ew_v.at[pl.ds(c * KD, KD)]
            pltpu.make_async_copy(src, din_sh.at[ni_v.at[c]], dsem).wait()
            pltpu.make_async_copy(src, dout_sh.at[no_v.at[c]], osem).wait()
        return carry
    lax.fori_loop(0, CHD // GRP, _grp_sc, 0)
    plsc.subcore_barrier()

    pltpu.sync_copy(din_sh.at[pl.ds(sid * RPT, RPT)], z_v)
    pltpu.sync_copy(z_v, din_out.at[pl.ds(cid * N_PAD + sid * RPT, RPT)])
    pltpu.sync_copy(dout_sh.at[pl.ds(sid * RPT, RPT)], z_v)
    pltpu.sync_copy(z_v, dout_out.at[pl.ds(cid * N_PAD + sid * RPT, RPT)])


@jax.jit
def _sc_degrees(ni2, no2, ew):
    return pl.kernel(
        _deg_body,
        out_type=[jax.ShapeDtypeStruct((NC * N_PAD,), jnp.float32),
                  jax.ShapeDtypeStruct((NC * N_PAD,), jnp.float32)],
        mesh=_sc_mesh(),
        scratch_types=[
            pltpu.VMEM((CHD, KD), jnp.int32),
            pltpu.VMEM((CHD, KD), jnp.int32),
            pltpu.VMEM((E // NW,), jnp.float32),
            pltpu.VMEM((RPT,), jnp.float32),
            pltpu.SemaphoreType.DMA,
            pltpu.SemaphoreType.DMA,
            pltpu.VMEM_SHARED((N_PAD,), jnp.float32),
            pltpu.VMEM_SHARED((N_PAD,), jnp.float32),
        ],
    )(ni2, no2, ew)


def _edge_body(ni_hbm, no_hbm, ew_hbm, x_hbm, din_hbm, acc_out,
               nir, nor, ewr, rinr, rb0, rb1, bufs,
               isems, gsems, rsems, ssems, acc_sh, rin_sh):
    cid = lax.axis_index("c")
    sid = lax.axis_index("s")
    wid = cid * NS + sid
    base = wid * EPTE

    pltpu.sync_copy(din_hbm.at[pl.ds(sid * RPT, RPT)], rb0)
    pltpu.sync_copy(din_hbm.at[pl.ds(N_PAD + sid * RPT, RPT)], rb1)

    def _rin(i, carry):
        sl = pl.ds(i * L, L)
        d = rb0[sl] + rb1[sl] + 1.0
        rb0[sl] = _rsqrt_nr(d)
        return carry
    lax.fori_loop(0, RPT // L, _rin, 0)
    pltpu.sync_copy(rb0, rin_sh.at[pl.ds(sid * RPT, RPT)])

    def _zr(r, carry):
        for q in range(D // L):
            bufs[0, r, pl.ds(q * L, L)] = jnp.zeros((L,), jnp.float32)
        return carry
    lax.fori_loop(0, KE, _zr, 0)
    for kk in range(RPT // KE):
        pltpu.sync_copy(bufs.at[0], acc_sh.at[pl.ds(sid * RPT + kk * KE, KE)])
    plsc.subcore_barrier()

    def _idx_start(t, b):
        sl = pl.ds(base + t * KE, KE)
        pltpu.async_copy(ni_hbm.at[sl], nir.at[b], isems.at[b])
        pltpu.async_copy(no_hbm.at[sl], nor.at[b], isems.at[b])
        pltpu.async_copy(ew_hbm.at[sl], ewr.at[b], isems.at[b])

    def _idx_wait(t, b):
        sl = pl.ds(base + t * KE, KE)
        pltpu.make_async_copy(ni_hbm.at[sl], nir.at[b], isems.at[b]).wait()
        pltpu.make_async_copy(no_hbm.at[sl], nor.at[b], isems.at[b]).wait()
        pltpu.make_async_copy(ew_hbm.at[sl], ewr.at[b], isems.at[b]).wait()

    def _gather_start(b):
        pltpu.async_copy(x_hbm.at[nir.at[b]], bufs.at[b], gsems.at[b])
        pltpu.async_copy(rin_sh.at[nir.at[b]], rinr.at[b], rsems.at[b])

    def _gather_wait(b):
        pltpu.make_async_copy(x_hbm.at[nir.at[b]], bufs.at[b],
                              gsems.at[b]).wait()
        pltpu.make_async_copy(rin_sh.at[nir.at[b]], rinr.at[b],
                              rsems.at[b]).wait()

    def _scatter_start(b):
        pltpu.async_copy(bufs.at[b], acc_sh.at[nor.at[b]], ssems.at[b],
                         add=True)

    def _scatter_wait(b):
        pltpu.make_async_copy(bufs.at[b], acc_sh.at[nor.at[b]],
                              ssems.at[b]).wait()

    def _scale(t, b):
        def _grp(g, gcarry):
            ewg = ewr[b, pl.ds(g * L, L)] * rinr[b, pl.ds(g * L, L)]
            for lane in range(L):
                w = _splat(ewg, lane)
                r = g * L + lane
                for q in range(D // L):
                    sl = pl.ds(q * L, L)
                    bufs[b, r, sl] = bufs[b, r, sl] * w
            return gcarry
        lax.fori_loop(0, KE // L, _grp, 0)

    for b in range(NBUF):
        _idx_start(b, b)
    for b in range(NBUF - 2):
        _idx_wait(b, b)
        _gather_start(b)

    def _slot(g, carry):
        for b in range(NBUF):
            t = g * NBUF + b
            pb = (b + NBUF - 1) % NBUF
            b3 = (b + NBUF - 2) % NBUF

            @pl.when(t > 0)
            def _():
                _scatter_wait(pb)

            @pl.when(jnp.logical_and(t > 0, t + NBUF - 1 < CHE))
            def _():
                _idx_start(t + NBUF - 1, pb)

            @pl.when(t + NBUF - 2 < CHE)
            def _():
                _idx_wait(t + NBUF - 2, b3)
                _gather_start(b3)

            _gather_wait(b)
            _scale(t, b)
            _scatter_start(b)
        return carry
    lax.fori_loop(0, CHE // NBUF, _slot, 0)
    _scatter_wait((CHE - 1) % NBUF)
    plsc.subcore_barrier()

    for kk in range(RPT // KE):
        sl = pl.ds(sid * RPT + kk * KE, KE)
        pltpu.sync_copy(acc_sh.at[sl], bufs.at[0])
        pltpu.sync_copy(bufs.at[0], acc_out.at[cid, sl])


@jax.jit
def _sc_edge_pass(ni, no, ew, x, din_f):
    return pl.kernel(
        _edge_body,
        out_type=jax.ShapeDtypeStruct((NC, N_PAD, D), jnp.float32),
        mesh=_sc_mesh(),
        scratch_types=[
            pltpu.VMEM((NBUF, KE), jnp.int32),
            pltpu.VMEM((NBUF, KE), jnp.int32),
            pltpu.VMEM((NBUF, KE), jnp.float32),
            pltpu.VMEM((NBUF, KE), jnp.float32),
            pltpu.VMEM((RPT,), jnp.float32),
            pltpu.VMEM((RPT,), jnp.float32),
            pltpu.VMEM((NBUF, KE, D), jnp.float32),
            pltpu.SemaphoreType.DMA((NBUF,)),
            pltpu.SemaphoreType.DMA((NBUF,)),
            pltpu.SemaphoreType.DMA((NBUF,)),
            pltpu.SemaphoreType.DMA((NBUF,)),
            pltpu.VMEM_SHARED((N_PAD, D), jnp.float32),
            pltpu.VMEM_SHARED((N_PAD,), jnp.float32),
        ],
    )(ni, no, ew, x, din_f)


def _out_body(accp_ref, x_ref, dinp_ref, doutp_ref, w_ref, b_ref, o_ref):
    rin = lax.rsqrt(dinp_ref[0, :] + dinp_ref[1, :] + 1.0)
    rout = lax.rsqrt(doutp_ref[0, :] + doutp_ref[1, :] + 1.0)
    u = (accp_ref[0] + accp_ref[1] + rin[:, None] * x_ref[...])
    u = u * rout[:, None]
    acc = lax.dot_general(u, w_ref[...], (((1,), (1,)), ((), ())),
                          preferred_element_type=jnp.float32)
    o_ref[...] = jnp.maximum(acc + b_ref[...], 0.0)


@jax.jit
def _tc_combine(acc_p, x_pad, din_p, dout_p, W, b2):
    blk = 1024
    return pl.pallas_call(
        _out_body,
        grid=(N_PAD // blk,),
        in_specs=[
            pl.BlockSpec((NC, blk, D), lambda i: (0, i, 0)),
            pl.BlockSpec((blk, D), lambda i: (i, 0)),
            pl.BlockSpec((NC, blk), lambda i: (0, i)),
            pl.BlockSpec((NC, blk), lambda i: (0, i)),
            pl.BlockSpec((D, D), lambda i: (0, 0)),
            pl.BlockSpec((1, D), lambda i: (0, 0)),
        ],
        out_specs=pl.BlockSpec((blk, D), lambda i: (i, 0)),
        out_shape=jax.ShapeDtypeStruct((N_PAD, D), jnp.float32),
    )(acc_p, x_pad, din_p, dout_p, W, b2)


def kernel(x, edge_index, edge_weight, W, b):
    ni = edge_index[:, 0]
    no = edge_index[:, 1]
    ept = E // NW

    ni2 = ni.reshape(NW, CHD, KD)
    no2 = no.reshape(NW, CHD, KD)

    pad_src = (jnp.arange(PADE, dtype=jnp.int32) * 41) % N
    pad_dst = (jnp.arange(PADE, dtype=jnp.int32) * 41) % N_PAD
    ni_p = jnp.concatenate(
        [ni.reshape(NW, ept),
         jnp.broadcast_to(pad_src, (NW, PADE))], axis=1).reshape(-1)
    no_p = jnp.concatenate(
        [no.reshape(NW, ept),
         jnp.broadcast_to(pad_dst, (NW, PADE))], axis=1).reshape(-1)
    ew_p = jnp.concatenate(
        [edge_weight.reshape(NW, ept),
         jnp.zeros((NW, PADE), jnp.float32)], axis=1).reshape(-1)

    x_pad = jnp.pad(x, ((0, N_PAD - N), (0, 0)))
    b2 = b.reshape(1, D)

    din_f, dout_f = _sc_degrees(ni2, no2, edge_weight)
    din_p = din_f.reshape(NC, N_PAD)
    dout_p = dout_f.reshape(NC, N_PAD)
    acc_p = _sc_edge_pass(ni_p, no_p, ew_p, x, din_f)
    out = _tc_combine(acc_p, x_pad, din_p, dout_p, W, b2)
    return out[:N]


# speedup vs baseline: 38.6448x; 1.0312x over previous
import jax
import jax.numpy as jnp
from jax import lax
from jax.experimental import pallas as pl
from jax.experimental.pallas import tpu as pltpu, tpu_sc as plsc

N = 10000
E = 320000
D = 128
N_PAD = 10240
NC, NS = 2, 16
NW = NC * NS
RPT = N_PAD // NS
L = 16

KD = 80
CHD = 125

KE = 80
CHE = 128
EPTE = KE * CHE
PADE = EPTE - E // NW
NBUF = 4


def _sc_mesh():
    return plsc.VectorSubcoreMesh(core_axis_name="c", subcore_axis_name="s")


def _splat(vec, lane):
    idx = jnp.full((L, 1), lane, jnp.int32)
    dnums = lax.GatherDimensionNumbers(
        offset_dims=(), collapsed_slice_dims=(0,), start_index_map=(0,))
    return lax.gather(vec, idx, dnums, (1,),
                      mode=lax.GatherScatterMode.PROMISE_IN_BOUNDS)


def _rsqrt_nr(d):
    i = lax.bitcast_convert_type(d, jnp.int32)
    i = jnp.int32(0x5F3759DF) - lax.shift_right_logical(i, 1)
    r = lax.bitcast_convert_type(i, jnp.float32)
    for _ in range(3):
        r = r * (1.5 - 0.5 * d * r * r)
    return r


def _deg_body(ni2_hbm, no2_hbm, ew_hbm, din_out, dout_out,
              ni_v, no_v, ew_v, z_v, dsem, osem, din_sh, dout_sh):
    cid = lax.axis_index("c")
    sid = lax.axis_index("s")
    wid = cid * NS + sid
    ept = E // NW

    pltpu.sync_copy(ni2_hbm.at[wid], ni_v)
    pltpu.sync_copy(no2_hbm.at[wid], no_v)
    pltpu.sync_copy(ew_hbm.at[pl.ds(wid * ept, ept)], ew_v)

    def _z(i, carry):
        z_v[pl.ds(i * L, L)] = jnp.zeros((L,), jnp.float32)
        return carry
    lax.fori_loop(0, RPT // L, _z, 0)
    pltpu.sync_copy(z_v, din_sh.at[pl.ds(sid * RPT, RPT)])
    pltpu.sync_copy(z_v, dout_sh.at[pl.ds(sid * RPT, RPT)])
    plsc.subcore_barrier()

    GRP = 25
    def _grp_sc(g, carry):
        for j in range(GRP):
            c = g * GRP + j
            src = ew_v.at[pl.ds(c * KD, KD)]
            pltpu.async_copy(src, din_sh.at[ni_v.at[c]], dsem, add=True)
            pltpu.async_copy(src, dout_sh.at[no_v.at[c]], osem, add=True)
        for j in range(GRP):
            c = g * GRP + j
            src = ew_v.at[pl.ds(c * KD, KD)]
            pltpu.make_async_copy(src, din_sh.at[ni_v.at[c]], dsem).wait()
            pltpu.make_async_copy(src, dout_sh.at[no_v.at[c]], osem).wait()
        return carry
    lax.fori_loop(0, CHD // GRP, _grp_sc, 0)
    plsc.subcore_barrier()

    pltpu.sync_copy(din_sh.at[pl.ds(sid * RPT, RPT)], z_v)
    pltpu.sync_copy(z_v, din_out.at[pl.ds(cid * N_PAD + sid * RPT, RPT)])
    pltpu.sync_copy(dout_sh.at[pl.ds(sid * RPT, RPT)], z_v)
    pltpu.sync_copy(z_v, dout_out.at[pl.ds(cid * N_PAD + sid * RPT, RPT)])


@jax.jit
def _sc_degrees(ni2, no2, ew):
    return pl.kernel(
        _deg_body,
        out_type=[jax.ShapeDtypeStruct((NC * N_PAD,), jnp.float32),
                  jax.ShapeDtypeStruct((NC * N_PAD,), jnp.float32)],
        mesh=_sc_mesh(),
        scratch_types=[
            pltpu.VMEM((CHD, KD), jnp.int32),
            pltpu.VMEM((CHD, KD), jnp.int32),
            pltpu.VMEM((E // NW,), jnp.float32),
            pltpu.VMEM((RPT,), jnp.float32),
            pltpu.SemaphoreType.DMA,
            pltpu.SemaphoreType.DMA,
            pltpu.VMEM_SHARED((N_PAD,), jnp.float32),
            pltpu.VMEM_SHARED((N_PAD,), jnp.float32),
        ],
    )(ni2, no2, ew)


def _edge_body(ni_hbm, no_hbm, ew_hbm, x_hbm, din_hbm, acc_out,
               nir, nor, ewr, rinr, rb0, rb1, bufs,
               isems, gsems, rsems, ssems, acc_sh, rin_sh):
    cid = lax.axis_index("c")
    sid = lax.axis_index("s")
    wid = cid * NS + sid
    base = wid * EPTE

    pltpu.sync_copy(din_hbm.at[pl.ds(sid * RPT, RPT)], rb0)
    pltpu.sync_copy(din_hbm.at[pl.ds(N_PAD + sid * RPT, RPT)], rb1)

    def _rin(i, carry):
        sl = pl.ds(i * L, L)
        d = rb0[sl] + rb1[sl] + 1.0
        rb0[sl] = _rsqrt_nr(d)
        return carry
    lax.fori_loop(0, RPT // L, _rin, 0)
    pltpu.sync_copy(rb0, rin_sh.at[pl.ds(sid * RPT, RPT)])

    def _zr(r, carry):
        for q in range(D // L):
            bufs[0, r, pl.ds(q * L, L)] = jnp.zeros((L,), jnp.float32)
        return carry
    lax.fori_loop(0, KE, _zr, 0)
    for kk in range(RPT // KE):
        pltpu.sync_copy(bufs.at[0], acc_sh.at[pl.ds(sid * RPT + kk * KE, KE)])
    plsc.subcore_barrier()

    NR = 2 * NBUF

    def _idx_start(t, r):
        sl = pl.ds(base + t * KE, KE)
        pltpu.async_copy(ni_hbm.at[sl], nir.at[r], isems.at[r])
        pltpu.async_copy(no_hbm.at[sl], nor.at[r], isems.at[r])
        pltpu.async_copy(ew_hbm.at[sl], ewr.at[r], isems.at[r])

    def _idx_wait(t, r):
        sl = pl.ds(base + t * KE, KE)
        pltpu.make_async_copy(ni_hbm.at[sl], nir.at[r], isems.at[r]).wait()
        pltpu.make_async_copy(no_hbm.at[sl], nor.at[r], isems.at[r]).wait()
        pltpu.make_async_copy(ew_hbm.at[sl], ewr.at[r], isems.at[r]).wait()

    def _gather_start(r, b):
        pltpu.async_copy(x_hbm.at[nir.at[r]], bufs.at[b], gsems.at[b])
        pltpu.async_copy(rin_sh.at[nir.at[r]], rinr.at[r], rsems.at[r])

    def _gather_wait(r, b):
        pltpu.make_async_copy(x_hbm.at[nir.at[r]], bufs.at[b],
                              gsems.at[b]).wait()
        pltpu.make_async_copy(rin_sh.at[nir.at[r]], rinr.at[r],
                              rsems.at[r]).wait()

    def _scatter_start(r, b):
        pltpu.async_copy(bufs.at[b], acc_sh.at[nor.at[r]], ssems.at[b],
                         add=True)

    def _scatter_wait(r, b):
        pltpu.make_async_copy(bufs.at[b], acc_sh.at[nor.at[r]],
                              ssems.at[b]).wait()

    def _scale(r, b):
        def _grp(g, gcarry):
            ewg = ewr[r, pl.ds(g * L, L)] * rinr[r, pl.ds(g * L, L)]
            for lane in range(L):
                w = _splat(ewg, lane)
                j = g * L + lane
                for q in range(D // L):
                    sl = pl.ds(q * L, L)
                    bufs[b, j, sl] = bufs[b, j, sl] * w
            return gcarry
        lax.fori_loop(0, KE // L, _grp, 0)

    for t0 in range(3):
        _idx_start(t0, t0)
    for t0 in range(2):
        _idx_wait(t0, t0)
        _gather_start(t0, t0)

    def _slot(g, carry):
        for b8 in range(NR):
            t = g * NR + b8
            b = b8 % NBUF
            rw = (b8 + NR - 2) % NR
            bw = (b8 + NBUF - 2) % NBUF
            r3 = (b8 + 3) % NR
            r2 = (b8 + 2) % NR
            b2 = (b8 + 2) % NBUF

            @pl.when(t >= 2)
            def _():
                _scatter_wait(rw, bw)

            @pl.when(t + 3 < CHE)
            def _():
                _idx_start(t + 3, r3)

            @pl.when(t + 2 < CHE)
            def _():
                _idx_wait(t + 2, r2)
                _gather_start(r2, b2)

            _gather_wait(b8, b)
            _scale(b8, b)
            _scatter_start(b8, b)
        return carry
    lax.fori_loop(0, CHE // NR, _slot, 0)
    _scatter_wait((CHE - 2) % NR, (CHE - 2) % NBUF)
    _scatter_wait((CHE - 1) % NR, (CHE - 1) % NBUF)
    plsc.subcore_barrier()

    for kk in range(RPT // KE):
        sl = pl.ds(sid * RPT + kk * KE, KE)
        pltpu.sync_copy(acc_sh.at[sl], bufs.at[0])
        pltpu.sync_copy(bufs.at[0], acc_out.at[cid, sl])


@jax.jit
def _sc_edge_pass(ni, no, ew, x, din_f):
    return pl.kernel(
        _edge_body,
        out_type=jax.ShapeDtypeStruct((NC, N_PAD, D), jnp.float32),
        mesh=_sc_mesh(),
        scratch_types=[
            pltpu.VMEM((2 * NBUF, KE), jnp.int32),
            pltpu.VMEM((2 * NBUF, KE), jnp.int32),
            pltpu.VMEM((2 * NBUF, KE), jnp.float32),
            pltpu.VMEM((2 * NBUF, KE), jnp.float32),
            pltpu.VMEM((RPT,), jnp.float32),
            pltpu.VMEM((RPT,), jnp.float32),
            pltpu.VMEM((NBUF, KE, D), jnp.float32),
            pltpu.SemaphoreType.DMA((2 * NBUF,)),
            pltpu.SemaphoreType.DMA((NBUF,)),
            pltpu.SemaphoreType.DMA((2 * NBUF,)),
            pltpu.SemaphoreType.DMA((NBUF,)),
            pltpu.VMEM_SHARED((N_PAD, D), jnp.float32),
            pltpu.VMEM_SHARED((N_PAD,), jnp.float32),
        ],
    )(ni, no, ew, x, din_f)


def _out_body(accp_ref, x_ref, dinp_ref, doutp_ref, w_ref, b_ref, o_ref):
    rin = lax.rsqrt(dinp_ref[0, :] + dinp_ref[1, :] + 1.0)
    rout = lax.rsqrt(doutp_ref[0, :] + doutp_ref[1, :] + 1.0)
    u = (accp_ref[0] + accp_ref[1] + rin[:, None] * x_ref[...])
    u = u * rout[:, None]
    acc = lax.dot_general(u, w_ref[...], (((1,), (1,)), ((), ())),
                          preferred_element_type=jnp.float32)
    o_ref[...] = jnp.maximum(acc + b_ref[...], 0.0)


@jax.jit
def _tc_combine(acc_p, x_pad, din_p, dout_p, W, b2):
    blk = 1024
    return pl.pallas_call(
        _out_body,
        grid=(N_PAD // blk,),
        in_specs=[
            pl.BlockSpec((NC, blk, D), lambda i: (0, i, 0)),
            pl.BlockSpec((blk, D), lambda i: (i, 0)),
            pl.BlockSpec((NC, blk), lambda i: (0, i)),
            pl.BlockSpec((NC, blk), lambda i: (0, i)),
            pl.BlockSpec((D, D), lambda i: (0, 0)),
            pl.BlockSpec((1, D), lambda i: (0, 0)),
        ],
        out_specs=pl.BlockSpec((blk, D), lambda i: (i, 0)),
        out_shape=jax.ShapeDtypeStruct((N_PAD, D), jnp.float32),
    )(acc_p, x_pad, din_p, dout_p, W, b2)


def kernel(x, edge_index, edge_weight, W, b):
    ni = edge_index[:, 0]
    no = edge_index[:, 1]
    ept = E // NW

    ni2 = ni.reshape(NW, CHD, KD)
    no2 = no.reshape(NW, CHD, KD)

    pad_src = (jnp.arange(PADE, dtype=jnp.int32) * 41) % N
    pad_dst = (jnp.arange(PADE, dtype=jnp.int32) * 41) % N_PAD
    ni_p = jnp.concatenate(
        [ni.reshape(NW, ept),
         jnp.broadcast_to(pad_src, (NW, PADE))], axis=1).reshape(-1)
    no_p = jnp.concatenate(
        [no.reshape(NW, ept),
         jnp.broadcast_to(pad_dst, (NW, PADE))], axis=1).reshape(-1)
    ew_p = jnp.concatenate(
        [edge_weight.reshape(NW, ept),
         jnp.zeros((NW, PADE), jnp.float32)], axis=1).reshape(-1)

    x_pad = jnp.pad(x, ((0, N_PAD - N), (0, 0)))
    b2 = b.reshape(1, D)

    din_f, dout_f = _sc_degrees(ni2, no2, edge_weight)
    din_p = din_f.reshape(NC, N_PAD)
    dout_p = dout_f.reshape(NC, N_PAD)
    acc_p = _sc_edge_pass(ni_p, no_p, ew_p, x, din_f)
    out = _tc_combine(acc_p, x_pad, din_p, dout_p, W, b2)
    return out[:N]
